# baseline (speedup 1.0000x reference)
import functools

import jax
import jax.numpy as jnp
from jax import lax
from jax.experimental import pallas as pl
from jax.experimental.pallas import tpu as pltpu
from jax.experimental.pallas import tpu_sc as plsc

B = 1024
N = 32768
L = 16

_NUM_CORES = 2
_NUM_SUBCORES = 16
_NW = _NUM_CORES * _NUM_SUBCORES
_ROWS_PER_W = B // _NW

BN = 16384
_NBLK = N // BN
_TOT = _ROWS_PER_W * _NBLK
_MW = BN // 4


def _masked_cumsum_body(x_hbm, m_hbm, out_hbm,
                        x0, x1, m0, m1, ov,
                        sx0, sx1, sm0, sm1):
    wid = lax.axis_index("s") * _NUM_CORES + lax.axis_index("c")
    base_row = wid * _ROWS_PER_W

    def x_at(b):
        row = base_row + b // _NBLK
        return x_hbm.at[row, pl.ds((b % _NBLK) * BN, BN)]

    def o_at(b):
        row = base_row + b // _NBLK
        return out_hbm.at[row, pl.ds((b % _NBLK) * BN, BN)]

    def m_at(b):
        row = base_row + b // _NBLK
        return m_hbm.at[row, pl.ds((b % _NBLK) * _MW, _MW)]

    def issue_in(b, xv, mv, sx, sm):
        @pl.when(b < _TOT)
        def _():
            pltpu.async_copy(x_at(b), xv, sx)
            pltpu.async_copy(m_at(b), mv, sm)

    def wait_in(xv, mv, sx, sm):
        pltpu.make_async_copy(x_hbm.at[0, pl.ds(0, BN)], xv, sx).wait()
        pltpu.make_async_copy(m_hbm.at[0, pl.ds(0, _MW)], mv, sm).wait()

    lane = lax.iota(jnp.int32, L)
    q4 = lax.shift_right_logical(lane, 2)
    sh = lax.shift_left(jnp.bitwise_and(lane, 3), 3)

    def step(b, xv, mv, carry):
        carry = jnp.where(b % _NBLK == 0, jnp.float32(0.0), carry)

        @plsc.parallel_loop(0, BN // L, carry=carry, unroll=8)
        def chunk(i, c):
            sl = pl.ds(i * L, L)
            mw = plsc.load_gather(mv, [q4 + i * 4])
            mf = jnp.bitwise_and(lax.shift_right_logical(mw, sh), 1)
            v = xv[sl] * mf.astype(jnp.float32)
            ov[sl] = plsc.cumsum(v) + c
            return c + jnp.sum(v)

        pltpu.sync_copy(ov, o_at(b))
        return chunk

    issue_in(0, x0, m0, sx0, sm0)

    def pair(t, carry):
        b0 = 2 * t
        b1 = 2 * t + 1
        wait_in(x0, m0, sx0, sm0)
        issue_in(b1, x1, m1, sx1, sm1)
        carry = step(b0, x0, m0, carry)
        wait_in(x1, m1, sx1, sm1)
        issue_in(b1 + 1, x0, m0, sx0, sm0)
        carry = step(b1, x1, m1, carry)
        return carry

    lax.fori_loop(0, _TOT // 2, pair, jnp.float32(0.0))


@jax.jit
def _masked_cumsum(x, m):
    mesh = plsc.VectorSubcoreMesh(core_axis_name="c", subcore_axis_name="s")
    fn = functools.partial(
        pl.kernel,
        mesh=mesh,
        out_type=jax.ShapeDtypeStruct((B, N), jnp.float32),
        scratch_types=[
            pltpu.VMEM((BN,), jnp.float32),
            pltpu.VMEM((BN,), jnp.float32),
            pltpu.VMEM((_MW,), jnp.int32),
            pltpu.VMEM((_MW,), jnp.int32),
            pltpu.VMEM((BN,), jnp.float32),
            pltpu.SemaphoreType.DMA,
            pltpu.SemaphoreType.DMA,
            pltpu.SemaphoreType.DMA,
            pltpu.SemaphoreType.DMA,
        ],
        compiler_params=pltpu.CompilerParams(needs_layout_passes=False),
    )(_masked_cumsum_body)
    return fn(x, m)


def kernel(x, mask):
    m32 = lax.bitcast_convert_type(
        mask.astype(jnp.uint8).reshape(B, N // 4, 4), jnp.int32)
    return _masked_cumsum(x, m32)


# speedup vs baseline: 2.7177x; 2.7177x over previous
import functools

import jax
import jax.numpy as jnp
from jax import lax
from jax.experimental import pallas as pl
from jax.experimental.pallas import tpu as pltpu
from jax.experimental.pallas import tpu_sc as plsc

B = 1024
N = 32768
L = 16

_NUM_CORES = 2
_NUM_SUBCORES = 16
_NW = _NUM_CORES * _NUM_SUBCORES
_ROWS_PER_W = B // _NW

BN = 16384
_NBLK = N // BN
_TOT = _ROWS_PER_W * _NBLK


def _masked_cumsum_body(x_hbm, m_hbm, out_hbm,
                        x0, x1, m0, m1, ov,
                        sx0, sx1, sm0, sm1):
    wid = lax.axis_index("s") * _NUM_CORES + lax.axis_index("c")
    base_row = wid * _ROWS_PER_W

    def hbm_at(ref, b):
        row = base_row + b // _NBLK
        off = (b % _NBLK) * BN
        return ref.at[row, pl.ds(off, BN)]

    def issue_in(b, xv, mv, sx, sm):
        @pl.when(b < _TOT)
        def _():
            pltpu.async_copy(hbm_at(x_hbm, b), xv, sx)
            pltpu.async_copy(hbm_at(m_hbm, b), mv, sm)

    def wait_in(xv, mv, sx, sm):
        pltpu.make_async_copy(x_hbm.at[0, pl.ds(0, BN)], xv, sx).wait()
        pltpu.make_async_copy(m_hbm.at[0, pl.ds(0, BN)], mv, sm).wait()

    def step(b, xv, mv, carry):
        carry = jnp.where(b % _NBLK == 0, jnp.float32(0.0), carry)

        @plsc.parallel_loop(0, BN // L, carry=carry, unroll=8)
        def chunk(i, c):
            sl = pl.ds(i * L, L)
            v = xv[sl] * mv[sl]
            ov[sl] = plsc.cumsum(v) + c
            return c + jnp.sum(v)

        pltpu.sync_copy(ov, hbm_at(out_hbm, b))
        return chunk

    issue_in(0, x0, m0, sx0, sm0)

    def pair(t, carry):
        b0 = 2 * t
        b1 = 2 * t + 1
        wait_in(x0, m0, sx0, sm0)
        issue_in(b1, x1, m1, sx1, sm1)
        carry = step(b0, x0, m0, carry)
        wait_in(x1, m1, sx1, sm1)
        issue_in(b1 + 1, x0, m0, sx0, sm0)
        carry = step(b1, x1, m1, carry)
        return carry

    lax.fori_loop(0, _TOT // 2, pair, jnp.float32(0.0))


@jax.jit
def _masked_cumsum(x, m):
    mesh = plsc.VectorSubcoreMesh(core_axis_name="c", subcore_axis_name="s")
    fn = functools.partial(
        pl.kernel,
        mesh=mesh,
        out_type=jax.ShapeDtypeStruct((B, N), jnp.float32),
        scratch_types=[
            pltpu.VMEM((BN,), jnp.float32),
            pltpu.VMEM((BN,), jnp.float32),
            pltpu.VMEM((BN,), jnp.float32),
            pltpu.VMEM((BN,), jnp.float32),
            pltpu.VMEM((BN,), jnp.float32),
            pltpu.SemaphoreType.DMA,
            pltpu.SemaphoreType.DMA,
            pltpu.SemaphoreType.DMA,
            pltpu.SemaphoreType.DMA,
        ],
        compiler_params=pltpu.CompilerParams(needs_layout_passes=False),
    )(_masked_cumsum_body)
    return fn(x, m)


def kernel(x, mask):
    return _masked_cumsum(x, mask.astype(jnp.float32))
